# Initial kernel scaffold; baseline (speedup 1.0000x reference)
import jax
import jax.numpy as jnp
from jax.experimental import pallas as pl


def kernel(x, weight, bias):
    raise NotImplementedError("write your pallas kernel here")


# speedup vs baseline: 1.1430x; 1.1430x over previous
import jax
import jax.numpy as jnp
from jax import lax
from jax.experimental import pallas as pl
from jax.experimental.pallas import tpu as pltpu


def _matvec_kernel(x_ref, w_ref, b_ref, o_ref):
    xb = x_ref[...].astype(jnp.bfloat16)
    acc = lax.dot_general(
        xb, w_ref[...],
        dimension_numbers=(((1,), (1,)), ((), ())),
        preferred_element_type=jnp.float32,
    )
    o_ref[...] = acc + b_ref[...]


def kernel(x, weight, bias):
    B, F = x.shape
    C, F_w = weight.shape
    assert F == F_w and bias.shape == (C,)

    C_pad = max(8, pl.cdiv(C, 8) * 8)
    w_p = jnp.pad(weight, ((0, C_pad - C), (0, 0))).astype(jnp.bfloat16)
    b_p = jnp.pad(bias, (0, C_pad - C)).reshape(1, C_pad)

    tm = 1024
    while B % tm != 0 and tm > 8:
        tm //= 2
    n_tiles = pl.cdiv(B, tm)
    B_pad = n_tiles * tm
    if B_pad != B:
        x = jnp.pad(x, ((0, B_pad - B), (0, 0)))

    out = pl.pallas_call(
        _matvec_kernel,
        out_shape=jax.ShapeDtypeStruct((B_pad, C_pad), jnp.float32),
        grid=(n_tiles,),
        in_specs=[
            pl.BlockSpec((tm, F), lambda i: (i, 0)),
            pl.BlockSpec((C_pad, F), lambda i: (0, 0)),
            pl.BlockSpec((1, C_pad), lambda i: (0, 0)),
        ],
        out_specs=pl.BlockSpec((tm, C_pad), lambda i: (i, 0)),
        compiler_params=pltpu.CompilerParams(
            dimension_semantics=("parallel",),
        ),
        cost_estimate=pl.CostEstimate(
            flops=2 * B_pad * F * C_pad,
            transcendentals=0,
            bytes_accessed=4 * (B_pad * F + B_pad * C_pad) + 2 * C_pad * F,
        ),
    )(x, w_p, b_p)
    return out[:B, :C]
